# baseline (speedup 1.0000x reference)
import jax
import jax.numpy as jnp
from jax.experimental import pallas as pl
from jax.experimental.pallas import tpu as pltpu

_PACK = 4


def _mlp_packed_kernel(xp_ref, xhp_ref, wx_ref, wxh_ref, w2_ref, w3_ref,
                       w4_ref, b1_ref, b2_ref, b3_ref, b4_ref, o_ref):
    h = jnp.dot(xp_ref[...], wx_ref[...], preferred_element_type=jnp.float32)
    h = h + jnp.dot(xhp_ref[...], wxh_ref[...],
                    preferred_element_type=jnp.float32)
    h = jax.nn.sigmoid(h + b1_ref[...])
    h = jax.nn.sigmoid(
        jnp.dot(h, w2_ref[...], preferred_element_type=jnp.float32)
        + b2_ref[...])
    h = jax.nn.sigmoid(
        jnp.dot(h, w3_ref[...], preferred_element_type=jnp.float32)
        + b3_ref[...])
    h = jax.nn.sigmoid(
        jnp.dot(h, w4_ref[...], preferred_element_type=jnp.float32)
        + b4_ref[...])
    o_ref[...] = h.astype(o_ref.dtype)


def kernel(x, x_hat, w1, b1, w2, b2, w3, b3, w4, b4, *, batch_tile=512):
    bs = x.shape[0]
    feat = x.shape[1] * x.shape[2]
    fp = _PACK * feat

    rows = bs // _PACK
    xp = x.reshape(rows, fp).astype(jnp.float32)
    xhp = x_hat.reshape(rows, fp).astype(jnp.float32)

    eye = jnp.eye(_PACK, dtype=jnp.float32)
    w1f = w1.astype(jnp.float32)
    wx = jnp.kron(eye, w1f[:feat])
    wxh = jnp.kron(eye, w1f[feat:])
    w2p = jnp.kron(eye, w2.astype(jnp.float32))
    w3p = jnp.kron(eye, w3.astype(jnp.float32))
    w4p = jnp.kron(eye, w4.astype(jnp.float32))
    b1p = jnp.tile(b1.astype(jnp.float32).reshape(1, -1), (1, _PACK))
    b2p = jnp.tile(b2.astype(jnp.float32).reshape(1, -1), (1, _PACK))
    b3p = jnp.tile(b3.astype(jnp.float32).reshape(1, -1), (1, _PACK))
    b4p = jnp.tile(b4.astype(jnp.float32).reshape(1, -1), (1, _PACK))

    tb = min(batch_tile, rows)
    pad = (-rows) % tb
    if pad:
        xp = jnp.pad(xp, ((0, pad), (0, 0)))
        xhp = jnp.pad(xhp, ((0, pad), (0, 0)))
    rows_p = rows + pad
    grid = rows_p // tb

    out = pl.pallas_call(
        _mlp_packed_kernel,
        out_shape=jax.ShapeDtypeStruct((rows_p, _PACK), jnp.float32),
        grid=(grid,),
        in_specs=[
            pl.BlockSpec((tb, fp), lambda i: (i, 0)),
            pl.BlockSpec((tb, fp), lambda i: (i, 0)),
            pl.BlockSpec(wx.shape, lambda i: (0, 0)),
            pl.BlockSpec(wxh.shape, lambda i: (0, 0)),
            pl.BlockSpec(w2p.shape, lambda i: (0, 0)),
            pl.BlockSpec(w3p.shape, lambda i: (0, 0)),
            pl.BlockSpec(w4p.shape, lambda i: (0, 0)),
            pl.BlockSpec(b1p.shape, lambda i: (0, 0)),
            pl.BlockSpec(b2p.shape, lambda i: (0, 0)),
            pl.BlockSpec(b3p.shape, lambda i: (0, 0)),
            pl.BlockSpec(b4p.shape, lambda i: (0, 0)),
        ],
        out_specs=pl.BlockSpec((tb, _PACK), lambda i: (i, 0)),
        compiler_params=pltpu.CompilerParams(
            dimension_semantics=("parallel",)),
    )(xp, xhp, wx, wxh, w2p, w3p, w4p, b1p, b2p, b3p, b4p)

    return out[:rows].reshape(bs, 1)


# speedup vs baseline: 8.4875x; 8.4875x over previous
import jax
import jax.numpy as jnp
from jax.experimental import pallas as pl
from jax.experimental.pallas import tpu as pltpu

_PACK = 4


def _mlp_packed_kernel(xp_ref, xhp_ref, wx_ref, wxh_ref, w2_ref, w3_ref,
                       w4_ref, b1_ref, b2_ref, b3_ref, b4_ref, o_ref):
    h = jnp.dot(xp_ref[...], wx_ref[...], preferred_element_type=jnp.float32)
    h = h + jnp.dot(xhp_ref[...], wxh_ref[...],
                    preferred_element_type=jnp.float32)
    h = jax.nn.sigmoid(h + b1_ref[...])
    h = jax.nn.sigmoid(
        jnp.dot(h, w2_ref[...], preferred_element_type=jnp.float32)
        + b2_ref[...])
    h = jax.nn.sigmoid(
        jnp.dot(h, w3_ref[...], preferred_element_type=jnp.float32)
        + b3_ref[...])
    h = jax.nn.sigmoid(
        jnp.dot(h, w4_ref[...], preferred_element_type=jnp.float32)
        + b4_ref[...])
    o_ref[...] = h.astype(o_ref.dtype)


def kernel(x, x_hat, w1, b1, w2, b2, w3, b3, w4, b4, *, batch_tile=512):
    bs = x.shape[0]
    feat = x.shape[1] * x.shape[2]
    fp = _PACK * feat

    rows = bs // _PACK
    xf = jax.lax.optimization_barrier(x.reshape(bs, feat).astype(jnp.float32))
    xhf = jax.lax.optimization_barrier(
        x_hat.reshape(bs, feat).astype(jnp.float32))
    xp = xf.reshape(rows, fp)
    xhp = xhf.reshape(rows, fp)

    eye = jnp.eye(_PACK, dtype=jnp.float32)
    w1f = w1.astype(jnp.float32)
    wx = jnp.kron(eye, w1f[:feat])
    wxh = jnp.kron(eye, w1f[feat:])
    w2p = jnp.kron(eye, w2.astype(jnp.float32))
    w3p = jnp.kron(eye, w3.astype(jnp.float32))
    w4p = jnp.kron(eye, w4.astype(jnp.float32))
    b1p = jnp.tile(b1.astype(jnp.float32).reshape(1, -1), (1, _PACK))
    b2p = jnp.tile(b2.astype(jnp.float32).reshape(1, -1), (1, _PACK))
    b3p = jnp.tile(b3.astype(jnp.float32).reshape(1, -1), (1, _PACK))
    b4p = jnp.tile(b4.astype(jnp.float32).reshape(1, -1), (1, _PACK))

    tb = min(batch_tile, rows)
    pad = (-rows) % tb
    if pad:
        xp = jnp.pad(xp, ((0, pad), (0, 0)))
        xhp = jnp.pad(xhp, ((0, pad), (0, 0)))
    rows_p = rows + pad
    grid = rows_p // tb

    out = pl.pallas_call(
        _mlp_packed_kernel,
        out_shape=jax.ShapeDtypeStruct((rows_p, _PACK), jnp.float32),
        grid=(grid,),
        in_specs=[
            pl.BlockSpec((tb, fp), lambda i: (i, 0)),
            pl.BlockSpec((tb, fp), lambda i: (i, 0)),
            pl.BlockSpec(wx.shape, lambda i: (0, 0)),
            pl.BlockSpec(wxh.shape, lambda i: (0, 0)),
            pl.BlockSpec(w2p.shape, lambda i: (0, 0)),
            pl.BlockSpec(w3p.shape, lambda i: (0, 0)),
            pl.BlockSpec(w4p.shape, lambda i: (0, 0)),
            pl.BlockSpec(b1p.shape, lambda i: (0, 0)),
            pl.BlockSpec(b2p.shape, lambda i: (0, 0)),
            pl.BlockSpec(b3p.shape, lambda i: (0, 0)),
            pl.BlockSpec(b4p.shape, lambda i: (0, 0)),
        ],
        out_specs=pl.BlockSpec((tb, _PACK), lambda i: (i, 0)),
        compiler_params=pltpu.CompilerParams(
            dimension_semantics=("parallel",)),
    )(xp, xhp, wx, wxh, w2p, w3p, w4p, b1p, b2p, b3p, b4p)

    return out[:rows].reshape(bs, 1)
